# baseline (speedup 1.0000x reference)
import functools

import jax
import jax.numpy as jnp
from jax import lax
from jax.experimental import pallas as pl
from jax.experimental.pallas import tpu as pltpu
from jax.experimental.pallas import tpu_sc as plsc

_L = 16


def _worker_ids(num_cores):
    return lax.axis_index("s") * num_cores + lax.axis_index("c")


def _compute_fid(bcol_hbm, xcol_hbm, bcol_v, xcol_v, idxf_v, n_rows,
                 base, b_per_w, chunk, bidx_row=None):
    pltpu.sync_copy(bcol_hbm.at[pl.ds(base, b_per_w)], bcol_v)
    pltpu.sync_copy(xcol_hbm.at[pl.ds(base, b_per_w)], xcol_v)
    for i in range(b_per_w // _L):
        bvec = bcol_v[pl.ds(i * _L, _L)]
        xvec = xcol_v[pl.ds(i * _L, _L)]
        flat = bvec * n_rows + xvec
        j, c = (i * _L) // chunk, (i * _L) % chunk
        idxf_v[j, pl.ds(c, _L)] = flat
        if bidx_row is not None:
            bidx_row[pl.ds(i * _L, _L)] = bvec.astype(jnp.float32)


@functools.lru_cache(maxsize=None)
def _build_main(n_rows: int, s_pad: int, b_per_w: int, n_chunk: int,
                chunk: int, num_cores: int):
    mesh = plsc.VectorSubcoreMesh(core_axis_name="c", subcore_axis_name="s")

    @functools.partial(
        pl.kernel,
        mesh=mesh,
        compiler_params=pltpu.CompilerParams(use_tc_tiling_on_sc=False),
        out_type=jax.ShapeDtypeStruct((6, s_pad), jnp.float32),
        scratch_types=[
            pltpu.VMEM((b_per_w,), jnp.int32),
            pltpu.VMEM((b_per_w,), jnp.int32),
            pltpu.VMEM((n_chunk, chunk), jnp.int32),
            pltpu.VMEM((6, b_per_w), jnp.float32),
            pltpu.SemaphoreType.DMA,
        ],
    )
    def main_kernel(bcol_hbm, xcol_hbm, boxes_hbm, scores_hbm, out_hbm,
                    bcol_v, xcol_v, idxf_v, out_v, sem):
        base = _worker_ids(num_cores) * b_per_w
        _compute_fid(bcol_hbm, xcol_hbm, bcol_v, xcol_v, idxf_v, n_rows,
                     base, b_per_w, chunk, bidx_row=out_v.at[0])
        copies = []
        for j in range(n_chunk):
            idxs = idxf_v.at[j]
            dst = pl.ds(j * chunk, chunk)
            copies.append(pltpu.async_copy(
                scores_hbm.at[idxs], out_v.at[5, dst], sem))
            for col in range(4):
                copies.append(pltpu.async_copy(
                    boxes_hbm.at[col].at[idxs], out_v.at[1 + col, dst], sem))
        for cp in copies:
            cp.wait()
        pltpu.sync_copy(out_v, out_hbm.at[:, pl.ds(base, b_per_w)])

    return main_kernel


@functools.lru_cache(maxsize=None)
def _build_planes(n_planes: int, n_rows: int, s_pad: int, b_per_w: int,
                  n_chunk: int, chunk: int, num_cores: int):
    mesh = plsc.VectorSubcoreMesh(core_axis_name="c", subcore_axis_name="s")

    @functools.partial(
        pl.kernel,
        mesh=mesh,
        compiler_params=pltpu.CompilerParams(use_tc_tiling_on_sc=False),
        out_type=jax.ShapeDtypeStruct((n_planes, s_pad), jnp.float32),
        scratch_types=[
            pltpu.VMEM((b_per_w,), jnp.int32),
            pltpu.VMEM((b_per_w,), jnp.int32),
            pltpu.VMEM((n_chunk, chunk), jnp.int32),
            pltpu.VMEM((n_planes, b_per_w), jnp.float32),
            pltpu.SemaphoreType.DMA,
        ],
    )
    def plane_kernel(bcol_hbm, xcol_hbm, planes_hbm, out_hbm,
                     bcol_v, xcol_v, idxf_v, out_v, sem):
        base = _worker_ids(num_cores) * b_per_w
        _compute_fid(bcol_hbm, xcol_hbm, bcol_v, xcol_v, idxf_v, n_rows,
                     base, b_per_w, chunk)
        copies = []
        for j in range(n_chunk):
            idxs = idxf_v.at[j]
            dst = pl.ds(j * chunk, chunk)
            for col in range(n_planes):
                copies.append(pltpu.async_copy(
                    planes_hbm.at[col].at[idxs], out_v.at[col, dst], sem))
        for cp in copies:
            cp.wait()
        pltpu.sync_copy(out_v, out_hbm.at[:, pl.ds(base, b_per_w)])

    return plane_kernel


def kernel(pred_boxes, pred_scores, pred_joints, selected_indexes):
    b, n = pred_boxes.shape[0], pred_boxes.shape[1]
    s = selected_indexes.shape[0]
    width_j = pred_joints.shape[2] * pred_joints.shape[3]

    info = plsc.get_sparse_core_info()
    nw = info.num_cores * info.num_subcores
    chunk = 80
    n_chunk = 2
    b_per_w = n_chunk * chunk
    s_pad = nw * b_per_w

    boxes_t = pred_boxes.transpose(2, 0, 1).reshape(4, b * n)
    scores_f = pred_scores.reshape(b * n)
    joints_t = pred_joints.transpose(2, 3, 0, 1).reshape(width_j, b * n)
    bcol = jnp.zeros((s_pad,), jnp.int32).at[:s].set(selected_indexes[:, 0])
    xcol = jnp.zeros((s_pad,), jnp.int32).at[:s].set(selected_indexes[:, 2])

    main_fn = _build_main(n, s_pad, b_per_w, n_chunk, chunk, info.num_cores)
    parts = [main_fn(bcol, xcol, boxes_t, scores_f)]

    n_groups = 4
    bounds = [round(g * width_j / n_groups) for g in range(n_groups + 1)]
    for lo, hi in zip(bounds[:-1], bounds[1:]):
        fn = _build_planes(hi - lo, n, s_pad, b_per_w, n_chunk, chunk,
                           info.num_cores)
        parts.append(fn(bcol, xcol, joints_t[lo:hi]))

    out_t = jnp.concatenate(parts, axis=0)
    return out_t[:, :s].T


# speedup vs baseline: 2.1123x; 2.1123x over previous
import functools

import jax
import jax.numpy as jnp
from jax import lax
from jax.experimental import pallas as pl
from jax.experimental.pallas import tpu as pltpu
from jax.experimental.pallas import tpu_sc as plsc

_L = 16


def _worker_ids(num_cores):
    return lax.axis_index("s") * num_cores + lax.axis_index("c")


def _compute_fid(bcol_hbm, xcol_hbm, bcol_v, xcol_v, idxf_v, n_rows,
                 base, b_per_w, chunk, bidx_row=None):
    pltpu.sync_copy(bcol_hbm.at[pl.ds(base, b_per_w)], bcol_v)
    pltpu.sync_copy(xcol_hbm.at[pl.ds(base, b_per_w)], xcol_v)
    for i in range(b_per_w // _L):
        bvec = bcol_v[pl.ds(i * _L, _L)]
        xvec = xcol_v[pl.ds(i * _L, _L)]
        flat = bvec * n_rows + xvec
        j, c = (i * _L) // chunk, (i * _L) % chunk
        idxf_v[j, pl.ds(c, _L)] = flat
        if bidx_row is not None:
            bidx_row[pl.ds(i * _L, _L)] = bvec.astype(jnp.float32)


@functools.lru_cache(maxsize=None)
def _build_main(n_rows: int, s_pad: int, b_per_w: int, n_chunk: int,
                chunk: int, num_cores: int):
    mesh = plsc.VectorSubcoreMesh(core_axis_name="c", subcore_axis_name="s")

    @functools.partial(
        pl.kernel,
        mesh=mesh,
        compiler_params=pltpu.CompilerParams(use_tc_tiling_on_sc=False),
        out_type=jax.ShapeDtypeStruct((6, s_pad), jnp.float32),
        scratch_types=[
            pltpu.VMEM((b_per_w,), jnp.int32),
            pltpu.VMEM((b_per_w,), jnp.int32),
            pltpu.VMEM((n_chunk, chunk), jnp.int32),
            pltpu.VMEM((6, b_per_w), jnp.float32),
            pltpu.SemaphoreType.DMA,
        ],
    )
    def main_kernel(bcol_hbm, xcol_hbm, boxes_hbm, scores_hbm, out_hbm,
                    bcol_v, xcol_v, idxf_v, out_v, sem):
        base = _worker_ids(num_cores) * b_per_w
        _compute_fid(bcol_hbm, xcol_hbm, bcol_v, xcol_v, idxf_v, n_rows,
                     base, b_per_w, chunk, bidx_row=out_v.at[0])
        copies = []
        for j in range(n_chunk):
            idxs = idxf_v.at[j]
            dst = pl.ds(j * chunk, chunk)
            copies.append(pltpu.async_copy(
                scores_hbm.at[idxs], out_v.at[5, dst], sem))
            for col in range(4):
                copies.append(pltpu.async_copy(
                    boxes_hbm.at[col].at[idxs], out_v.at[1 + col, dst], sem))
        for cp in copies:
            cp.wait()
        pltpu.sync_copy(out_v, out_hbm.at[:, pl.ds(base, b_per_w)])

    return main_kernel


@functools.lru_cache(maxsize=None)
def _build_planes(n_planes: int, n_rows: int, s_pad: int, b_per_w: int,
                  n_chunk: int, chunk: int, num_cores: int):
    mesh = plsc.VectorSubcoreMesh(core_axis_name="c", subcore_axis_name="s")

    @functools.partial(
        pl.kernel,
        mesh=mesh,
        compiler_params=pltpu.CompilerParams(use_tc_tiling_on_sc=False),
        out_type=jax.ShapeDtypeStruct((n_planes, s_pad), jnp.float32),
        scratch_types=[
            pltpu.VMEM((b_per_w,), jnp.int32),
            pltpu.VMEM((b_per_w,), jnp.int32),
            pltpu.VMEM((n_chunk, chunk), jnp.int32),
            pltpu.VMEM((n_planes, b_per_w), jnp.float32),
            pltpu.SemaphoreType.DMA,
        ],
    )
    def plane_kernel(bcol_hbm, xcol_hbm, planes_hbm, out_hbm,
                     bcol_v, xcol_v, idxf_v, out_v, sem):
        base = _worker_ids(num_cores) * b_per_w
        _compute_fid(bcol_hbm, xcol_hbm, bcol_v, xcol_v, idxf_v, n_rows,
                     base, b_per_w, chunk)
        copies = []
        for j in range(n_chunk):
            idxs = idxf_v.at[j]
            dst = pl.ds(j * chunk, chunk)
            for col in range(n_planes):
                copies.append(pltpu.async_copy(
                    planes_hbm.at[col].at[idxs], out_v.at[col, dst], sem))
        for cp in copies:
            cp.wait()
        pltpu.sync_copy(out_v, out_hbm.at[:, pl.ds(base, b_per_w)])

    return plane_kernel


def kernel(pred_boxes, pred_scores, pred_joints, selected_indexes):
    b, n = pred_boxes.shape[0], pred_boxes.shape[1]
    s = selected_indexes.shape[0]
    width_j = pred_joints.shape[2] * pred_joints.shape[3]

    info = plsc.get_sparse_core_info()
    nw = info.num_cores * info.num_subcores
    chunk = 80
    n_chunk = 2
    b_per_w = n_chunk * chunk
    s_pad = nw * b_per_w

    boxes_t = pred_boxes.transpose(2, 0, 1).reshape(4, b * n)
    scores_f = pred_scores.reshape(b * n)
    joints_t = pred_joints.transpose(2, 3, 0, 1).reshape(width_j, b * n)
    bcol = jnp.zeros((s_pad,), jnp.int32).at[:s].set(selected_indexes[:, 0])
    xcol = jnp.zeros((s_pad,), jnp.int32).at[:s].set(selected_indexes[:, 2])

    main_fn = _build_main(n, s_pad, b_per_w, n_chunk, chunk, info.num_cores)
    head = main_fn(bcol, xcol, boxes_t, scores_f)
    joints_fn = _build_planes(width_j, n, s_pad, b_per_w, n_chunk, chunk,
                              info.num_cores)
    tail = joints_fn(bcol, xcol, joints_t)

    out_t = jnp.concatenate([head, tail], axis=0)
    return out_t[:, :s].T


# speedup vs baseline: 2.6637x; 1.2610x over previous
import functools

import jax
import jax.numpy as jnp
from jax import lax
from jax.experimental import pallas as pl
from jax.experimental.pallas import tpu as pltpu
from jax.experimental.pallas import tpu_sc as plsc

_L = 16


def _worker_ids(num_cores):
    return lax.axis_index("s") * num_cores + lax.axis_index("c")


def _compute_fid(bcol_hbm, xcol_hbm, bcol_v, xcol_v, idxf_v, n_rows,
                 base, b_per_w, chunk, bidx_row=None):
    pltpu.sync_copy(bcol_hbm.at[pl.ds(base, b_per_w)], bcol_v)
    pltpu.sync_copy(xcol_hbm.at[pl.ds(base, b_per_w)], xcol_v)
    for i in range(b_per_w // _L):
        bvec = bcol_v[pl.ds(i * _L, _L)]
        xvec = xcol_v[pl.ds(i * _L, _L)]
        flat = bvec * n_rows + xvec
        j, c = (i * _L) // chunk, (i * _L) % chunk
        idxf_v[j, pl.ds(c, _L)] = flat
        if bidx_row is not None:
            bidx_row[pl.ds(i * _L, _L)] = bvec.astype(jnp.float32)


@functools.lru_cache(maxsize=None)
def _build_gather(n_rows: int, width_j: int, s_pad: int, b_per_w: int,
                  n_chunk: int, chunk: int, num_cores: int):
    n_cols = 1 + 4 + 1 + width_j
    mesh = plsc.VectorSubcoreMesh(core_axis_name="c", subcore_axis_name="s")

    @functools.partial(
        pl.kernel,
        mesh=mesh,
        compiler_params=pltpu.CompilerParams(use_tc_tiling_on_sc=False),
        out_type=jax.ShapeDtypeStruct((n_cols, s_pad), jnp.float32),
        scratch_types=[
            pltpu.VMEM((b_per_w,), jnp.int32),
            pltpu.VMEM((b_per_w,), jnp.int32),
            pltpu.VMEM((n_chunk, chunk), jnp.int32),
            pltpu.VMEM((n_cols, b_per_w), jnp.float32),
            pltpu.SemaphoreType.DMA,
        ],
    )
    def gather_kernel(bcol_hbm, xcol_hbm, boxes_hbm, scores_hbm, joints_hbm,
                      out_hbm, bcol_v, xcol_v, idxf_v, out_v, sem):
        base = _worker_ids(num_cores) * b_per_w
        _compute_fid(bcol_hbm, xcol_hbm, bcol_v, xcol_v, idxf_v, n_rows,
                     base, b_per_w, chunk, bidx_row=out_v.at[0])
        copies = []
        for j in range(n_chunk):
            idxs = idxf_v.at[j]
            dst = pl.ds(j * chunk, chunk)
            copies.append(pltpu.async_copy(
                scores_hbm.at[idxs], out_v.at[5, dst], sem))
            for col in range(4):
                copies.append(pltpu.async_copy(
                    boxes_hbm.at[col].at[idxs], out_v.at[1 + col, dst], sem))
            for col in range(width_j):
                copies.append(pltpu.async_copy(
                    joints_hbm.at[col].at[idxs], out_v.at[6 + col, dst], sem))
        for cp in copies:
            cp.wait()
        pltpu.sync_copy(out_v, out_hbm.at[:, pl.ds(base, b_per_w)])

    return gather_kernel


def kernel(pred_boxes, pred_scores, pred_joints, selected_indexes):
    b, n = pred_boxes.shape[0], pred_boxes.shape[1]
    s = selected_indexes.shape[0]
    width_j = pred_joints.shape[2] * pred_joints.shape[3]

    info = plsc.get_sparse_core_info()
    nw = info.num_cores * info.num_subcores
    chunk = 80
    n_chunk = 2
    b_per_w = n_chunk * chunk
    s_pad = nw * b_per_w

    boxes_t = pred_boxes.transpose(2, 0, 1).reshape(4, b * n)
    scores_f = pred_scores.reshape(b * n)
    joints_t = pred_joints.transpose(2, 3, 0, 1).reshape(width_j, b * n)
    bcol = jnp.zeros((s_pad,), jnp.int32).at[:s].set(selected_indexes[:, 0])
    xcol = jnp.zeros((s_pad,), jnp.int32).at[:s].set(selected_indexes[:, 2])

    fn = _build_gather(n, width_j, s_pad, b_per_w, n_chunk, chunk,
                       info.num_cores)
    out_t = fn(bcol, xcol, boxes_t, scores_f, joints_t)
    return out_t[:, :s].T
